# Initial kernel scaffold; baseline (speedup 1.0000x reference)
import jax
import jax.numpy as jnp
from jax.experimental import pallas as pl


def kernel(x, edge_index, Ws0, bs0, Wn0, bn0, g0, be0, Ws1, bs1, Wn1, bn1, g1, be1, Ws2, bs2, Wn2, bn2):
    raise NotImplementedError("write your pallas kernel here")


# speedup vs baseline: 7.5616x; 7.5616x over previous
import functools

import jax
import jax.numpy as jnp
from jax import lax
from jax.experimental import pallas as pl
from jax.experimental.pallas import tpu as pltpu
from jax.experimental.pallas import tpu_sc as plsc

_N = 10000
_E = 320000
_NW = 32
_EPW = _E // _NW
_C = 80
_NCHUNK = _EPW // _C
_NP = 10240
_RPT = _NP // 16


def _make_edge_agg(width):
    mesh = plsc.VectorSubcoreMesh(core_axis_name="c", subcore_axis_name="s")

    @functools.partial(
        pl.kernel,
        mesh=mesh,
        out_type=jax.ShapeDtypeStruct((2, _NP, width), jnp.float32),
        scratch_types=[
            pltpu.VMEM((_NCHUNK, _C), jnp.int32),
            pltpu.VMEM((_NCHUNK, _C), jnp.int32),
            pltpu.VMEM((_C, width), jnp.float32),
            pltpu.VMEM_SHARED((_NP, width), jnp.float32),
            pltpu.SemaphoreType.DMA,
        ],
        compiler_params=pltpu.CompilerParams(use_tc_tiling_on_sc=False),
    )
    def agg_kernel(z_hbm, src_hbm, dst_hbm, zeros_hbm, out_hbm,
                   src_v, dst_v, rows_v, acc_sh, sem):
        cid = lax.axis_index("c")
        sid = lax.axis_index("s")
        wid = sid * 2 + cid
        pltpu.sync_copy(zeros_hbm, acc_sh.at[pl.ds(sid * _RPT, _RPT)])
        pltpu.sync_copy(src_hbm.at[wid], src_v)
        pltpu.sync_copy(dst_hbm.at[wid], dst_v)
        plsc.subcore_barrier()

        def body(j, carry):
            pltpu.async_copy(z_hbm.at[src_v.at[j]], rows_v, sem).wait()
            pltpu.sync_copy(rows_v, acc_sh.at[dst_v.at[j]], add=True)
            return carry

        lax.fori_loop(0, _NCHUNK, body, 0, unroll=False)
        plsc.subcore_barrier()
        pltpu.sync_copy(acc_sh.at[pl.ds(sid * _RPT, _RPT)],
                        out_hbm.at[cid, pl.ds(sid * _RPT, _RPT)])

    return agg_kernel


_agg144 = _make_edge_agg(144)
_agg128 = _make_edge_agg(128)
_agg64 = _make_edge_agg(64)


def _tc_z0(x_ref, wn_ref, out_ref):
    out_ref[:, :128] = jnp.dot(x_ref[...], wn_ref[...],
                               preferred_element_type=jnp.float32)
    col = lax.broadcasted_iota(jnp.int32, (_N, 16), 1)
    out_ref[:, 128:144] = jnp.where(col == 0, 1.0, 0.0).astype(jnp.float32)


def _tc_layer0(x_ref, agg_ref, ws_ref, bs_ref, bn_ref, g_ref, be_ref, wn1_ref,
               h1_ref, z1_ref, inv_ref):
    agg = agg_ref[0, :_N] + agg_ref[1, :_N]
    deg = jnp.sum(agg[:, 128:144], axis=1, keepdims=True)
    invd = 1.0 / jnp.maximum(deg, 1.0)
    pre = (jnp.dot(x_ref[...], ws_ref[...], preferred_element_type=jnp.float32)
           + bs_ref[...] + bn_ref[...] + agg[:, :128] * invd)
    mu = jnp.mean(pre, axis=0, keepdims=True)
    var = jnp.mean((pre - mu) * (pre - mu), axis=0, keepdims=True)
    h = g_ref[...] * (pre - mu) * lax.rsqrt(var + 1e-5) + be_ref[...]
    h = jnp.maximum(h, 0.0)
    h1_ref[...] = h
    z1_ref[...] = jnp.dot(h, wn1_ref[...], preferred_element_type=jnp.float32)
    inv_ref[...] = invd


def _tc_layer1(h1_ref, agg_ref, inv_ref, ws_ref, bs_ref, bn_ref, g_ref, be_ref,
               wn2_ref, h2_ref, z2_ref):
    agg = agg_ref[0, :_N] + agg_ref[1, :_N]
    pre = (jnp.dot(h1_ref[...], ws_ref[...], preferred_element_type=jnp.float32)
           + bs_ref[...] + bn_ref[...] + agg * inv_ref[...])
    mu = jnp.mean(pre, axis=0, keepdims=True)
    var = jnp.mean((pre - mu) * (pre - mu), axis=0, keepdims=True)
    h = g_ref[...] * (pre - mu) * lax.rsqrt(var + 1e-5) + be_ref[...]
    h = jnp.maximum(h, 0.0)
    h2_ref[...] = h
    z2_ref[...] = jnp.dot(h, wn2_ref[...], preferred_element_type=jnp.float32)


def _tc_layer2(h2_ref, agg_ref, inv_ref, ws_ref, bs_ref, bn_ref, out_ref):
    agg = agg_ref[0, :_N] + agg_ref[1, :_N]
    out_ref[...] = (jnp.dot(h2_ref[...], ws_ref[...],
                            preferred_element_type=jnp.float32)
                    + bs_ref[...] + bn_ref[...] + agg * inv_ref[...])


def kernel(x, edge_index, Ws0, bs0, Wn0, bn0, g0, be0,
           Ws1, bs1, Wn1, bn1, g1, be1, Ws2, bs2, Wn2, bn2):
    f32 = jnp.float32
    src = edge_index[0].reshape(_NW, _NCHUNK, _C)
    dst = edge_index[1].reshape(_NW, _NCHUNK, _C)

    z0 = pl.pallas_call(
        _tc_z0,
        out_shape=jax.ShapeDtypeStruct((_N, 144), f32),
    )(x, Wn0)

    agg0 = _agg144(z0, src, dst, jnp.zeros((_RPT, 144), f32))

    h1, z1, invd = pl.pallas_call(
        _tc_layer0,
        out_shape=(
            jax.ShapeDtypeStruct((_N, 128), f32),
            jax.ShapeDtypeStruct((_N, 128), f32),
            jax.ShapeDtypeStruct((_N, 1), f32),
        ),
    )(x, agg0, Ws0, bs0.reshape(1, 128), bn0.reshape(1, 128),
      g0.reshape(1, 128), be0.reshape(1, 128), Wn1)

    agg1 = _agg128(z1, src, dst, jnp.zeros((_RPT, 128), f32))

    h2, z2 = pl.pallas_call(
        _tc_layer1,
        out_shape=(
            jax.ShapeDtypeStruct((_N, 128), f32),
            jax.ShapeDtypeStruct((_N, 64), f32),
        ),
    )(h1, agg1, invd, Ws1, bs1.reshape(1, 128), bn1.reshape(1, 128),
      g1.reshape(1, 128), be1.reshape(1, 128), Wn2)

    agg2 = _agg64(z2, src, dst, jnp.zeros((_RPT, 64), f32))

    out = pl.pallas_call(
        _tc_layer2,
        out_shape=jax.ShapeDtypeStruct((_N, 64), f32),
    )(h2, agg2, invd, Ws2, bs2.reshape(1, 64), bn2.reshape(1, 64))

    return out


# speedup vs baseline: 11.2648x; 1.4897x over previous
import functools

import jax
import jax.numpy as jnp
from jax import lax
from jax.experimental import pallas as pl
from jax.experimental.pallas import tpu as pltpu
from jax.experimental.pallas import tpu_sc as plsc

_N = 10000
_E = 320000
_NW = 32
_EPW = _E // _NW
_NP = 10240
_RPT = _NP // 16


def _make_edge_agg(width, chunk, nbuf):
    nchunk = _EPW // chunk
    groups = nchunk // nbuf
    tail = nchunk % nbuf
    mesh = plsc.VectorSubcoreMesh(core_axis_name="c", subcore_axis_name="s")

    @functools.partial(
        pl.kernel,
        mesh=mesh,
        out_type=jax.ShapeDtypeStruct((2, _NP, width), jnp.float32),
        scratch_types=[
            pltpu.VMEM((nchunk, chunk), jnp.int32),
            pltpu.VMEM((nchunk, chunk), jnp.int32),
            pltpu.VMEM((nbuf, chunk, width), jnp.float32),
            pltpu.VMEM_SHARED((_NP, width), jnp.float32),
            pltpu.SemaphoreType.DMA((nbuf,)),
            pltpu.SemaphoreType.DMA((nbuf,)),
        ],
        compiler_params=pltpu.CompilerParams(use_tc_tiling_on_sc=False),
    )
    def agg_kernel(z_hbm, src_hbm, dst_hbm, zeros_hbm, out_hbm,
                   src_v, dst_v, rows_v, acc_sh, gsem, ssem):
        cid = lax.axis_index("c")
        sid = lax.axis_index("s")
        wid = sid * 2 + cid
        pltpu.sync_copy(zeros_hbm, acc_sh.at[pl.ds(sid * _RPT, _RPT)])
        pltpu.sync_copy(src_hbm.at[wid], src_v)
        pltpu.sync_copy(dst_hbm.at[wid], dst_v)
        plsc.subcore_barrier()

        def start_gather(j, b):
            pltpu.async_copy(z_hbm.at[src_v.at[j]], rows_v.at[b], gsem.at[b])

        def wait_gather(j, b):
            pltpu.make_async_copy(z_hbm.at[src_v.at[j]], rows_v.at[b],
                                  gsem.at[b]).wait()

        for b in range(nbuf):
            start_gather(b, b)

        def group(g, carry):
            for b in range(nbuf):
                j = g * nbuf + b
                wait_gather(j, b)
                pltpu.async_copy(rows_v.at[b], acc_sh.at[dst_v.at[j]],
                                 ssem.at[b], add=True)
                pltpu.make_async_copy(rows_v.at[b], acc_sh.at[dst_v.at[j]],
                                      ssem.at[b]).wait()

                @pl.when(j + nbuf < nchunk)
                def _():
                    start_gather(j + nbuf, b)
            return carry

        lax.fori_loop(0, groups, group, 0, unroll=False)
        for t in range(tail):
            j = groups * nbuf + t
            wait_gather(j, t)
            pltpu.sync_copy(rows_v.at[t], acc_sh.at[dst_v.at[j]], add=True)
        plsc.subcore_barrier()
        pltpu.sync_copy(acc_sh.at[pl.ds(sid * _RPT, _RPT)],
                        out_hbm.at[cid, pl.ds(sid * _RPT, _RPT)])

    return agg_kernel


_agg144 = _make_edge_agg(144, 40, 2)
_agg128 = _make_edge_agg(128, 80, 2)
_agg64 = _make_edge_agg(64, 80, 4)


def _tc_z0(x_ref, wn_ref, out_ref):
    out_ref[:, :128] = jnp.dot(x_ref[...], wn_ref[...],
                               preferred_element_type=jnp.float32)
    col = lax.broadcasted_iota(jnp.int32, (_N, 16), 1)
    out_ref[:, 128:144] = jnp.where(col == 0, 1.0, 0.0).astype(jnp.float32)


def _tc_layer0(x_ref, agg_ref, ws_ref, bs_ref, bn_ref, g_ref, be_ref, wn1_ref,
               h1_ref, z1_ref, inv_ref):
    agg = agg_ref[0, :_N] + agg_ref[1, :_N]
    deg = jnp.sum(agg[:, 128:144], axis=1, keepdims=True)
    invd = 1.0 / jnp.maximum(deg, 1.0)
    pre = (jnp.dot(x_ref[...], ws_ref[...], preferred_element_type=jnp.float32)
           + bs_ref[...] + bn_ref[...] + agg[:, :128] * invd)
    mu = jnp.mean(pre, axis=0, keepdims=True)
    var = jnp.mean((pre - mu) * (pre - mu), axis=0, keepdims=True)
    h = g_ref[...] * (pre - mu) * lax.rsqrt(var + 1e-5) + be_ref[...]
    h = jnp.maximum(h, 0.0)
    h1_ref[...] = h
    z1_ref[...] = jnp.dot(h, wn1_ref[...], preferred_element_type=jnp.float32)
    inv_ref[...] = invd


def _tc_layer1(h1_ref, agg_ref, inv_ref, ws_ref, bs_ref, bn_ref, g_ref, be_ref,
               wn2_ref, h2_ref, z2_ref):
    agg = agg_ref[0, :_N] + agg_ref[1, :_N]
    pre = (jnp.dot(h1_ref[...], ws_ref[...], preferred_element_type=jnp.float32)
           + bs_ref[...] + bn_ref[...] + agg * inv_ref[...])
    mu = jnp.mean(pre, axis=0, keepdims=True)
    var = jnp.mean((pre - mu) * (pre - mu), axis=0, keepdims=True)
    h = g_ref[...] * (pre - mu) * lax.rsqrt(var + 1e-5) + be_ref[...]
    h = jnp.maximum(h, 0.0)
    h2_ref[...] = h
    z2_ref[...] = jnp.dot(h, wn2_ref[...], preferred_element_type=jnp.float32)


def _tc_layer2(h2_ref, agg_ref, inv_ref, ws_ref, bs_ref, bn_ref, out_ref):
    agg = agg_ref[0, :_N] + agg_ref[1, :_N]
    out_ref[...] = (jnp.dot(h2_ref[...], ws_ref[...],
                            preferred_element_type=jnp.float32)
                    + bs_ref[...] + bn_ref[...] + agg * inv_ref[...])


def kernel(x, edge_index, Ws0, bs0, Wn0, bn0, g0, be0,
           Ws1, bs1, Wn1, bn1, g1, be1, Ws2, bs2, Wn2, bn2):
    f32 = jnp.float32
    src40 = edge_index[0].reshape(_NW, _EPW // 40, 40)
    dst40 = edge_index[1].reshape(_NW, _EPW // 40, 40)
    src80 = edge_index[0].reshape(_NW, _EPW // 80, 80)
    dst80 = edge_index[1].reshape(_NW, _EPW // 80, 80)

    z0 = pl.pallas_call(
        _tc_z0,
        out_shape=jax.ShapeDtypeStruct((_N, 144), f32),
    )(x, Wn0)

    agg0 = _agg144(z0, src40, dst40, jnp.zeros((_RPT, 144), f32))

    h1, z1, invd = pl.pallas_call(
        _tc_layer0,
        out_shape=(
            jax.ShapeDtypeStruct((_N, 128), f32),
            jax.ShapeDtypeStruct((_N, 128), f32),
            jax.ShapeDtypeStruct((_N, 1), f32),
        ),
    )(x, agg0, Ws0, bs0.reshape(1, 128), bn0.reshape(1, 128),
      g0.reshape(1, 128), be0.reshape(1, 128), Wn1)

    agg1 = _agg128(z1, src80, dst80, jnp.zeros((_RPT, 128), f32))

    h2, z2 = pl.pallas_call(
        _tc_layer1,
        out_shape=(
            jax.ShapeDtypeStruct((_N, 128), f32),
            jax.ShapeDtypeStruct((_N, 64), f32),
        ),
    )(h1, agg1, invd, Ws1, bs1.reshape(1, 128), bn1.reshape(1, 128),
      g1.reshape(1, 128), be1.reshape(1, 128), Wn2)

    agg2 = _agg64(z2, src80, dst80, jnp.zeros((_RPT, 64), f32))

    out = pl.pallas_call(
        _tc_layer2,
        out_shape=jax.ShapeDtypeStruct((_N, 64), f32),
    )(h2, agg2, invd, Ws2, bs2.reshape(1, 64), bn2.reshape(1, 64))

    return out


# speedup vs baseline: 13.1222x; 1.1649x over previous
import functools

import jax
import jax.numpy as jnp
from jax import lax
from jax.experimental import pallas as pl
from jax.experimental.pallas import tpu as pltpu
from jax.experimental.pallas import tpu_sc as plsc

_N = 10000
_E = 320000
_NW = 32
_EPW = _E // _NW
_NP = 10240
_RPT = _NP // 16


def _make_edge_agg(width, chunk, nbuf):
    nchunk = _EPW // chunk
    groups = nchunk // nbuf
    tail = nchunk % nbuf
    mesh = plsc.VectorSubcoreMesh(core_axis_name="c", subcore_axis_name="s")

    @functools.partial(
        pl.kernel,
        mesh=mesh,
        out_type=jax.ShapeDtypeStruct((2, _NP, width), jnp.float32),
        scratch_types=[
            pltpu.VMEM((nchunk, chunk), jnp.int32),
            pltpu.VMEM((nchunk, chunk), jnp.int32),
            pltpu.VMEM((nbuf, chunk, width), jnp.float32),
            pltpu.VMEM_SHARED((_NP, width), jnp.float32),
            pltpu.SemaphoreType.DMA((nbuf,)),
            pltpu.SemaphoreType.DMA((nbuf,)),
        ],
        compiler_params=pltpu.CompilerParams(use_tc_tiling_on_sc=False),
    )
    def agg_kernel(z_hbm, src_hbm, dst_hbm, zeros_hbm, out_hbm,
                   src_v, dst_v, rows_v, acc_sh, gsem, ssem):
        cid = lax.axis_index("c")
        sid = lax.axis_index("s")
        wid = sid * 2 + cid
        pltpu.sync_copy(zeros_hbm, acc_sh.at[pl.ds(sid * _RPT, _RPT)])
        pltpu.sync_copy(src_hbm.at[wid], src_v)
        pltpu.sync_copy(dst_hbm.at[wid], dst_v)
        plsc.subcore_barrier()

        def start_gather(j, b):
            pltpu.async_copy(z_hbm.at[src_v.at[j]], rows_v.at[b], gsem.at[b])

        def wait_gather(j, b):
            pltpu.make_async_copy(z_hbm.at[src_v.at[j]], rows_v.at[b],
                                  gsem.at[b]).wait()

        for b in range(nbuf):
            start_gather(b, b)

        def group(g, carry):
            for b in range(nbuf):
                j = g * nbuf + b
                wait_gather(j, b)
                pltpu.async_copy(rows_v.at[b], acc_sh.at[dst_v.at[j]],
                                 ssem.at[b], add=True)
                pltpu.make_async_copy(rows_v.at[b], acc_sh.at[dst_v.at[j]],
                                      ssem.at[b]).wait()

                @pl.when(j + nbuf < nchunk)
                def _():
                    start_gather(j + nbuf, b)
            return carry

        lax.fori_loop(0, groups, group, 0, unroll=False)
        for t in range(tail):
            j = groups * nbuf + t
            wait_gather(j, t)
            pltpu.sync_copy(rows_v.at[t], acc_sh.at[dst_v.at[j]], add=True)
        plsc.subcore_barrier()
        pltpu.sync_copy(acc_sh.at[pl.ds(sid * _RPT, _RPT)],
                        out_hbm.at[cid, pl.ds(sid * _RPT, _RPT)])

    return agg_kernel


_agg144 = _make_edge_agg(144, 40, 3)
_agg128 = _make_edge_agg(128, 40, 5)
_agg64 = _make_edge_agg(64, 80, 4)


def _tc_z0(x_ref, wn_ref, out_ref):
    out_ref[:, :128] = jnp.dot(x_ref[...], wn_ref[...],
                               preferred_element_type=jnp.float32)
    col = lax.broadcasted_iota(jnp.int32, (_N, 16), 1)
    out_ref[:, 128:144] = jnp.where(col == 0, 1.0, 0.0).astype(jnp.float32)


def _tc_layer0(x_ref, agg_ref, ws_ref, bs_ref, bn_ref, g_ref, be_ref, wn1_ref,
               h1_ref, z1_ref, inv_ref):
    agg = agg_ref[0, :_N] + agg_ref[1, :_N]
    deg = jnp.sum(agg[:, 128:144], axis=1, keepdims=True)
    invd = 1.0 / jnp.maximum(deg, 1.0)
    pre = (jnp.dot(x_ref[...], ws_ref[...], preferred_element_type=jnp.float32)
           + bs_ref[...] + bn_ref[...] + agg[:, :128] * invd)
    mu = jnp.mean(pre, axis=0, keepdims=True)
    var = jnp.mean((pre - mu) * (pre - mu), axis=0, keepdims=True)
    h = g_ref[...] * (pre - mu) * lax.rsqrt(var + 1e-5) + be_ref[...]
    h = jnp.maximum(h, 0.0)
    h1_ref[...] = h
    z1_ref[...] = jnp.dot(h, wn1_ref[...], preferred_element_type=jnp.float32)
    inv_ref[...] = invd


def _tc_layer1(h1_ref, agg_ref, inv_ref, ws_ref, bs_ref, bn_ref, g_ref, be_ref,
               wn2_ref, h2_ref, z2_ref):
    agg = agg_ref[0, :_N] + agg_ref[1, :_N]
    pre = (jnp.dot(h1_ref[...], ws_ref[...], preferred_element_type=jnp.float32)
           + bs_ref[...] + bn_ref[...] + agg * inv_ref[...])
    mu = jnp.mean(pre, axis=0, keepdims=True)
    var = jnp.mean((pre - mu) * (pre - mu), axis=0, keepdims=True)
    h = g_ref[...] * (pre - mu) * lax.rsqrt(var + 1e-5) + be_ref[...]
    h = jnp.maximum(h, 0.0)
    h2_ref[...] = h
    z2_ref[...] = jnp.dot(h, wn2_ref[...], preferred_element_type=jnp.float32)


def _tc_layer2(h2_ref, agg_ref, inv_ref, ws_ref, bs_ref, bn_ref, out_ref):
    agg = agg_ref[0, :_N] + agg_ref[1, :_N]
    out_ref[...] = (jnp.dot(h2_ref[...], ws_ref[...],
                            preferred_element_type=jnp.float32)
                    + bs_ref[...] + bn_ref[...] + agg * inv_ref[...])


def kernel(x, edge_index, Ws0, bs0, Wn0, bn0, g0, be0,
           Ws1, bs1, Wn1, bn1, g1, be1, Ws2, bs2, Wn2, bn2):
    f32 = jnp.float32
    src40 = edge_index[0].reshape(_NW, _EPW // 40, 40)
    dst40 = edge_index[1].reshape(_NW, _EPW // 40, 40)
    src80 = edge_index[0].reshape(_NW, _EPW // 80, 80)
    dst80 = edge_index[1].reshape(_NW, _EPW // 80, 80)

    z0 = pl.pallas_call(
        _tc_z0,
        out_shape=jax.ShapeDtypeStruct((_N, 144), f32),
    )(x, Wn0)

    agg0 = _agg144(z0, src40, dst40, jnp.zeros((_RPT, 144), f32))

    h1, z1, invd = pl.pallas_call(
        _tc_layer0,
        out_shape=(
            jax.ShapeDtypeStruct((_N, 128), f32),
            jax.ShapeDtypeStruct((_N, 128), f32),
            jax.ShapeDtypeStruct((_N, 1), f32),
        ),
    )(x, agg0, Ws0, bs0.reshape(1, 128), bn0.reshape(1, 128),
      g0.reshape(1, 128), be0.reshape(1, 128), Wn1)

    agg1 = _agg128(z1, src40, dst40, jnp.zeros((_RPT, 128), f32))

    h2, z2 = pl.pallas_call(
        _tc_layer1,
        out_shape=(
            jax.ShapeDtypeStruct((_N, 128), f32),
            jax.ShapeDtypeStruct((_N, 64), f32),
        ),
    )(h1, agg1, invd, Ws1, bs1.reshape(1, 128), bn1.reshape(1, 128),
      g1.reshape(1, 128), be1.reshape(1, 128), Wn2)

    agg2 = _agg64(z2, src80, dst80, jnp.zeros((_RPT, 64), f32))

    out = pl.pallas_call(
        _tc_layer2,
        out_shape=jax.ShapeDtypeStruct((_N, 64), f32),
    )(h2, agg2, invd, Ws2, bs2.reshape(1, 64), bn2.reshape(1, 64))

    return out


# speedup vs baseline: 15.0839x; 1.1495x over previous
import functools

import jax
import jax.numpy as jnp
from jax import lax
from jax.experimental import pallas as pl
from jax.experimental.pallas import tpu as pltpu
from jax.experimental.pallas import tpu_sc as plsc

_N = 10000
_E = 320000
_NW = 32
_EPW = _E // _NW
_NP = 10240
_RPT = _NP // 16
_DC = 120


def _make_edge_agg(width, chunk, nbuf, with_deg=False, out_width=None):
    ow = out_width or width
    nchunk = _EPW // chunk
    groups = nchunk // nbuf
    tail = nchunk % nbuf
    if with_deg:
        assert _DC == nbuf * chunk
        dgroups = _EPW // _DC
        dtail = _EPW - dgroups * _DC
    mesh = plsc.VectorSubcoreMesh(core_axis_name="c", subcore_axis_name="s")

    out_type = [jax.ShapeDtypeStruct((2, _NP, ow), jnp.float32)]
    scratch = [
        pltpu.VMEM((_EPW,), jnp.int32),
        pltpu.VMEM((_EPW,), jnp.int32),
        pltpu.VMEM((nbuf, chunk, width), jnp.float32),
        pltpu.VMEM_SHARED((_NP, width), jnp.float32),
        pltpu.SemaphoreType.DMA((nbuf,)),
        pltpu.SemaphoreType.DMA((nbuf,)),
    ]
    if with_deg:
        out_type.append(jax.ShapeDtypeStruct((2, _NP, 16), jnp.float32))
        scratch += [
            pltpu.VMEM((_DC, 16), jnp.float32),
            pltpu.VMEM_SHARED((_NP, 16), jnp.float32),
            pltpu.SemaphoreType.DMA,
        ]

    @functools.partial(
        pl.kernel,
        mesh=mesh,
        out_type=tuple(out_type),
        scratch_types=scratch,
        compiler_params=pltpu.CompilerParams(use_tc_tiling_on_sc=False),
    )
    def agg_kernel(z_hbm, ei_hbm, zeros_hbm, *rest):
        if with_deg:
            (zeros16_hbm, out_hbm, dout_hbm, src_v, dst_v, rows_v, acc_sh,
             gsem, ssem, ones_v, dacc_sh, dsem) = rest
        else:
            out_hbm, src_v, dst_v, rows_v, acc_sh, gsem, ssem = rest
        cid = lax.axis_index("c")
        sid = lax.axis_index("s")
        wid = sid * 2 + cid
        pltpu.sync_copy(zeros_hbm, acc_sh.at[pl.ds(sid * _RPT, _RPT)])
        pltpu.sync_copy(ei_hbm.at[0, pl.ds(wid * _EPW, _EPW)], src_v)
        pltpu.sync_copy(ei_hbm.at[1, pl.ds(wid * _EPW, _EPW)], dst_v)
        if with_deg:
            pltpu.sync_copy(zeros16_hbm, dacc_sh.at[pl.ds(sid * _RPT, _RPT)])
            e0 = jnp.where(lax.iota(jnp.int32, 16) == 0, 1.0, 0.0)
            for r in range(_DC):
                ones_v[r, :] = e0
        plsc.subcore_barrier()

        def sidx(j):
            return src_v.at[pl.ds(j * chunk, chunk)]

        def didx(j):
            return dst_v.at[pl.ds(j * chunk, chunk)]

        def start_gather(j, b):
            pltpu.async_copy(z_hbm.at[sidx(j)], rows_v.at[b], gsem.at[b])

        def wait_gather(j, b):
            pltpu.make_async_copy(z_hbm.at[sidx(j)], rows_v.at[b],
                                  gsem.at[b]).wait()

        def deg_slice(g):
            return dacc_sh.at[dst_v.at[pl.ds(g * _DC, _DC)]]

        for b in range(nbuf):
            start_gather(b, b)

        def group(g, carry):
            for b in range(nbuf):
                j = g * nbuf + b
                if with_deg and b == 0:
                    @pl.when(g > 0)
                    def _():
                        pltpu.make_async_copy(ones_v, deg_slice(g - 1),
                                              dsem).wait()
                    pltpu.async_copy(ones_v, deg_slice(g), dsem, add=True)
                wait_gather(j, b)
                pltpu.async_copy(rows_v.at[b], acc_sh.at[didx(j)],
                                 ssem.at[b], add=True)
                pltpu.make_async_copy(rows_v.at[b], acc_sh.at[didx(j)],
                                      ssem.at[b]).wait()

                @pl.when(j + nbuf < nchunk)
                def _():
                    start_gather(j + nbuf, b)
            return carry

        lax.fori_loop(0, groups, group, 0, unroll=False)
        for t in range(tail):
            j = groups * nbuf + t
            wait_gather(j, t)
            pltpu.sync_copy(rows_v.at[t], acc_sh.at[didx(j)], add=True)
        if with_deg:
            pltpu.make_async_copy(ones_v, deg_slice(dgroups - 1), dsem).wait()
            if dtail:
                pltpu.sync_copy(
                    ones_v.at[pl.ds(0, dtail)],
                    dacc_sh.at[dst_v.at[pl.ds(dgroups * _DC, dtail)]],
                    add=True)
        plsc.subcore_barrier()
        rows = pl.ds(sid * _RPT, _RPT)
        if ow == width:
            pltpu.sync_copy(acc_sh.at[rows], out_hbm.at[cid, rows])
        else:
            pltpu.sync_copy(acc_sh.at[rows],
                            out_hbm.at[cid, rows, pl.ds(0, width)])
        if with_deg:
            pltpu.sync_copy(dacc_sh.at[rows], dout_hbm.at[cid, rows])

    return agg_kernel


_agg0 = _make_edge_agg(128, 40, 3, with_deg=True)
_agg1 = _make_edge_agg(128, 40, 5)
_agg2 = _make_edge_agg(64, 80, 6, out_width=128)


def _tc_z0(x_ref, wn_ref, out_ref):
    out_ref[...] = jnp.dot(x_ref[...], wn_ref[...],
                           preferred_element_type=jnp.float32)


def _tc_layer0(x_ref, agg_ref, deg_ref, ws_ref, bs_ref, bn_ref, g_ref, be_ref,
               wn1_ref, h1_ref, z1_ref, inv_ref):
    agg = agg_ref[0, :_N] + agg_ref[1, :_N]
    deg = jnp.sum(deg_ref[0, :_N] + deg_ref[1, :_N], axis=1, keepdims=True)
    invd = 1.0 / jnp.maximum(deg, 1.0)
    pre = (jnp.dot(x_ref[...], ws_ref[...], preferred_element_type=jnp.float32)
           + bs_ref[...] + bn_ref[...] + agg * invd)
    mu = jnp.mean(pre, axis=0, keepdims=True)
    var = jnp.mean((pre - mu) * (pre - mu), axis=0, keepdims=True)
    h = g_ref[...] * (pre - mu) * lax.rsqrt(var + 1e-5) + be_ref[...]
    h = jnp.maximum(h, 0.0)
    h1_ref[...] = h
    z1_ref[...] = jnp.dot(h, wn1_ref[...], preferred_element_type=jnp.float32)
    inv_ref[...] = invd


def _tc_layer1(h1_ref, agg_ref, inv_ref, ws_ref, bs_ref, bn_ref, g_ref, be_ref,
               wn2_ref, h2_ref, z2_ref):
    agg = agg_ref[0, :_N] + agg_ref[1, :_N]
    pre = (jnp.dot(h1_ref[...], ws_ref[...], preferred_element_type=jnp.float32)
           + bs_ref[...] + bn_ref[...] + agg * inv_ref[...])
    mu = jnp.mean(pre, axis=0, keepdims=True)
    var = jnp.mean((pre - mu) * (pre - mu), axis=0, keepdims=True)
    h = g_ref[...] * (pre - mu) * lax.rsqrt(var + 1e-5) + be_ref[...]
    h = jnp.maximum(h, 0.0)
    h2_ref[...] = h
    z2_ref[...] = jnp.dot(h, wn2_ref[...], preferred_element_type=jnp.float32)


def _tc_layer2(h2_ref, agg_ref, inv_ref, ws_ref, bs_ref, bn_ref, out_ref):
    agg = agg_ref[0, :_N, :64] + agg_ref[1, :_N, :64]
    out_ref[...] = (jnp.dot(h2_ref[...], ws_ref[...],
                            preferred_element_type=jnp.float32)
                    + bs_ref[...] + bn_ref[...] + agg * inv_ref[...])


def kernel(x, edge_index, Ws0, bs0, Wn0, bn0, g0, be0,
           Ws1, bs1, Wn1, bn1, g1, be1, Ws2, bs2, Wn2, bn2):
    f32 = jnp.float32
    zeros128 = jnp.zeros((_RPT, 128), f32)

    z0 = pl.pallas_call(
        _tc_z0,
        out_shape=jax.ShapeDtypeStruct((_N, 128), f32),
    )(x, Wn0)

    agg0, deg0 = _agg0(z0, edge_index, zeros128, jnp.zeros((_RPT, 16), f32))

    h1, z1, invd = pl.pallas_call(
        _tc_layer0,
        out_shape=(
            jax.ShapeDtypeStruct((_N, 128), f32),
            jax.ShapeDtypeStruct((_N, 128), f32),
            jax.ShapeDtypeStruct((_N, 1), f32),
        ),
    )(x, agg0, deg0, Ws0, bs0.reshape(1, 128), bn0.reshape(1, 128),
      g0.reshape(1, 128), be0.reshape(1, 128), Wn1)

    (agg1,) = _agg1(z1, edge_index, zeros128)

    h2, z2 = pl.pallas_call(
        _tc_layer1,
        out_shape=(
            jax.ShapeDtypeStruct((_N, 128), f32),
            jax.ShapeDtypeStruct((_N, 64), f32),
        ),
    )(h1, agg1, invd, Ws1, bs1.reshape(1, 128), bn1.reshape(1, 128),
      g1.reshape(1, 128), be1.reshape(1, 128), Wn2)

    (agg2,) = _agg2(z2, edge_index, jnp.zeros((_RPT, 64), f32))

    out = pl.pallas_call(
        _tc_layer2,
        out_shape=jax.ShapeDtypeStruct((_N, 64), f32),
    )(h2, agg2, invd, Ws2, bs2.reshape(1, 64), bn2.reshape(1, 64))

    return out
